# baseline (speedup 1.0000x reference)
import functools

import jax
import jax.numpy as jnp
from jax import lax
from jax.experimental import pallas as pl
from jax.experimental.pallas import tpu as pltpu
from jax.experimental.pallas import tpu_sc as plsc


@functools.lru_cache(maxsize=None)
def _make_sc_gather(V, D, B):
    info = plsc.get_sparse_core_info()
    NC, NS, L = info.num_cores, info.num_subcores, info.num_lanes
    NW = NC * NS
    assert D % L == 0 and B % (8 * NW) == 0
    b_per_w = B // NW
    mesh = plsc.VectorSubcoreMesh(core_axis_name="c", subcore_axis_name="s")

    @functools.partial(
        pl.kernel,
        mesh=mesh,
        out_type=jax.ShapeDtypeStruct((B, D), jnp.float32),
        scratch_types=[
            pltpu.VMEM((b_per_w,), jnp.int32),
            pltpu.VMEM((b_per_w, D), jnp.float32),
            pltpu.SemaphoreType.DMA,
        ],
        compiler_params=pltpu.CompilerParams(use_tc_tiling_on_sc=False),
    )
    def sc_gather(table_hbm, idx_hbm, out_hbm, idx_v, rows_v, sem):
        wid = lax.axis_index("s") * NC + lax.axis_index("c")
        base = wid * b_per_w
        pltpu.sync_copy(idx_hbm.at[pl.ds(base, b_per_w)], idx_v)
        pltpu.async_copy(table_hbm.at[idx_v], rows_v, sem).wait()
        pltpu.sync_copy(rows_v, out_hbm.at[pl.ds(base, b_per_w)])

    return sc_gather


def _tc_body(hid_ref, wt_ref, b_ref, out_ref, ls_ref, *, BT, NV):
    j = pl.program_id(1)

    @pl.when(j == 0)
    def _stats():
        hid = hid_ref[...]

        def step(k, carry):
            m, s = carry
            logits = (
                jnp.dot(hid, wt_ref[k], preferred_element_type=jnp.float32)
                + b_ref[k]
            )
            cm = jnp.max(logits, axis=1, keepdims=True)
            m2 = jnp.maximum(m, cm)
            s2 = s * jnp.exp(m - m2) + jnp.sum(
                jnp.exp(logits - m2), axis=1, keepdims=True
            )
            return m2, s2

        m0 = jnp.full((BT, 1), -1e30, jnp.float32)
        s0 = jnp.zeros((BT, 1), jnp.float32)
        m, s = lax.fori_loop(0, NV, step, (m0, s0))
        ls_ref[...] = m + jnp.log(s)

    logits = (
        jnp.dot(hid_ref[...], wt_ref[j], preferred_element_type=jnp.float32)
        + b_ref[j]
    )
    out_ref[...] = logits - ls_ref[...]


@functools.lru_cache(maxsize=None)
def _make_tc_logsoftmax(B, V, D, BT, VC):
    NV = -(-V // VC)
    body = functools.partial(_tc_body, BT=BT, NV=NV)
    return pl.pallas_call(
        body,
        grid=(B // BT, NV),
        in_specs=[
            pl.BlockSpec((BT, D), lambda i, j: (i, 0)),
            pl.BlockSpec((NV, D, VC), lambda i, j: (0, 0, 0)),
            pl.BlockSpec((NV, 1, VC), lambda i, j: (0, 0, 0)),
        ],
        out_specs=pl.BlockSpec((BT, VC), lambda i, j: (i, j)),
        out_shape=jax.ShapeDtypeStruct((B, V), jnp.float32),
        scratch_shapes=[pltpu.VMEM((BT, 1), jnp.float32)],
    )


def kernel(center_word_index, emb_table, W, b):
    V, D = emb_table.shape
    (B,) = center_word_index.shape
    BT = 256
    VC = 2048
    NV = -(-V // VC)
    VPAD = NV * VC

    idx = center_word_index.astype(jnp.int32)
    hidden = _make_sc_gather(V, D, B)(emb_table, idx)

    wt = jnp.pad(W.T, ((0, 0), (0, VPAD - V)))
    wt3 = wt.reshape(D, NV, VC).transpose(1, 0, 2)
    b3 = jnp.pad(b, (0, VPAD - V), constant_values=-1e30).reshape(NV, 1, VC)

    return _make_tc_logsoftmax(B, V, D, BT, VC)(hidden, wt3, b3)
